# baseline (speedup 1.0000x reference)
import functools

import jax
import jax.numpy as jnp
from jax import lax
from jax.experimental import pallas as pl
from jax.experimental.pallas import tpu as pltpu
from jax.experimental.pallas import tpu_sc as plsc

NC = 2
NS = 16
L = 16
NW = NC * NS


def _mf_body(u_hbm, i_hbm, W_hbm, H_hbm, out_hbm,
             uidx, iidx, urows, vrows, outv, sem_u, sem_v):
    wid = lax.axis_index("s") * NC + lax.axis_index("c")
    bpw = u_hbm.shape[0] // NW
    base = wid * bpw

    pltpu.sync_copy(u_hbm.at[pl.ds(base, bpw)], uidx)
    pltpu.sync_copy(i_hbm.at[pl.ds(base, bpw)], iidx)
    cu = pltpu.async_copy(W_hbm.at[uidx], urows, sem_u)
    cv = pltpu.async_copy(H_hbm.at[iidx], vrows, sem_v)
    cu.wait()
    cv.wait()

    lanes = lax.iota(jnp.int32, L)

    def block(b, carry):
        rows = b * L + lanes
        acc = jnp.zeros((L,), jnp.float32)
        for k in range(L):
            col = jnp.full((L,), k, jnp.int32)
            ug = plsc.load_gather(urows, [rows, col])
            vg = plsc.load_gather(vrows, [rows, col])
            acc = acc + ug * vg
        outv[pl.ds(b * L, L)] = 1.0 / (1.0 + jnp.exp(-acc))
        return carry

    lax.fori_loop(0, bpw // L, block, 0)
    pltpu.sync_copy(outv, out_hbm.at[pl.ds(base, bpw)])


def kernel(x, W, H):
    B = x.shape[0]
    u = x[:, 0].astype(jnp.int32)
    i = x[:, 1].astype(jnp.int32)
    bpw = B // NW
    mesh = plsc.VectorSubcoreMesh(
        core_axis_name="c", subcore_axis_name="s",
        num_cores=NC, num_subcores=NS)
    f = pl.kernel(
        _mf_body,
        out_type=jax.ShapeDtypeStruct((B,), jnp.float32),
        mesh=mesh,
        compiler_params=pltpu.CompilerParams(needs_layout_passes=False,
                                             use_tc_tiling_on_sc=False),
        scratch_types=[
            pltpu.VMEM((bpw,), jnp.int32),
            pltpu.VMEM((bpw,), jnp.int32),
            pltpu.VMEM((bpw, L), jnp.float32),
            pltpu.VMEM((bpw, L), jnp.float32),
            pltpu.VMEM((bpw,), jnp.float32),
            pltpu.SemaphoreType.DMA,
            pltpu.SemaphoreType.DMA,
        ],
    )
    return f(u, i, W, H)


# speedup vs baseline: 5.7074x; 5.7074x over previous
import functools

import jax
import jax.numpy as jnp
from jax import lax
from jax.experimental import pallas as pl
from jax.experimental.pallas import tpu as pltpu
from jax.experimental.pallas import tpu_sc as plsc

NC = 2
NS = 16
L = 16
NW = NC * NS
K = 16
G = 4


def _mf_body(u_hbm, i_hbm, wt_hbm, ht_hbm, out_hbm,
             uidx, iidx, bufw, bufh, prod, outv,
             sem_i, sem_w0, sem_w1, sem_h0, sem_h1):
    wid = lax.axis_index("s") * NC + lax.axis_index("c")
    bpw = u_hbm.shape[0] // NW
    base = wid * bpw
    ngrp = bpw // G

    cu = pltpu.async_copy(u_hbm.at[pl.ds(base, bpw)], uidx.at[pl.ds(0, bpw)],
                          sem_i)
    ci = pltpu.async_copy(i_hbm.at[pl.ds(base, bpw)], iidx.at[pl.ds(0, bpw)],
                          sem_i)
    cu.wait()
    ci.wait()
    uidx[pl.ds(bpw, L)] = jnp.zeros((L,), jnp.int32)
    iidx[pl.ds(bpw, L)] = jnp.zeros((L,), jnp.int32)

    lanes = lax.iota(jnp.int32, L)

    def fire(g, s, sem_w, sem_h):
        uu = uidx[pl.ds(g * G, L)]
        vv = iidx[pl.ds(g * G, L)]
        for t in range(G):
            cw = pl.multiple_of((uu[t] >> 7) * 128, 128)
            ch = pl.multiple_of((vv[t] >> 7) * 128, 128)
            pltpu.async_copy(wt_hbm.at[:, pl.ds(cw, 128)], bufw.at[s, t],
                             sem_w)
            pltpu.async_copy(ht_hbm.at[:, pl.ds(ch, 128)], bufh.at[s, t],
                             sem_h)

    def drain(s, sem_w, sem_h):
        for t in range(G):
            pltpu.make_async_copy(wt_hbm.at[:, pl.ds(0, 128)], bufw.at[s, t],
                                  sem_w).wait()
            pltpu.make_async_copy(ht_hbm.at[:, pl.ds(0, 128)], bufh.at[s, t],
                                  sem_h).wait()

    def process(g, s):
        uu = uidx[pl.ds(g * G, L)]
        vv = iidx[pl.ds(g * G, L)]
        for t in range(G):
            ou = jnp.full((L,), uu[t] & 127, jnp.int32)
            ov = jnp.full((L,), vv[t] & 127, jnp.int32)
            wcol = plsc.load_gather(bufw.at[s, t], [lanes, ou])
            hcol = plsc.load_gather(bufh.at[s, t], [lanes, ov])
            plsc.store_scatter(prod, [lanes * bpw + (g * G + t)], wcol * hcol)

    fire(0, 0, sem_w0, sem_h0)

    def body(gg, carry):
        g0 = 2 * gg
        fire(g0 + 1, 1, sem_w1, sem_h1)
        drain(0, sem_w0, sem_h0)
        process(g0, 0)

        @pl.when(gg < ngrp // 2 - 1)
        def _():
            fire(g0 + 2, 0, sem_w0, sem_h0)

        drain(1, sem_w1, sem_h1)
        process(g0 + 1, 1)
        return carry

    lax.fori_loop(0, ngrp // 2, body, 0)

    for c in range(bpw // L):
        acc = jnp.zeros((L,), jnp.float32)
        for k in range(K):
            acc += prod[pl.ds(k * bpw + c * L, L)]
        outv[pl.ds(c * L, L)] = 1.0 / (1.0 + jnp.exp(-acc))

    pltpu.sync_copy(outv, out_hbm.at[pl.ds(base, bpw)])


def kernel(x, W, H):
    B = x.shape[0]
    u = x[:, 0].astype(jnp.int32)
    i = x[:, 1].astype(jnp.int32)
    bpw = B // NW
    mesh = plsc.VectorSubcoreMesh(
        core_axis_name="c", subcore_axis_name="s",
        num_cores=NC, num_subcores=NS)
    f = pl.kernel(
        _mf_body,
        out_type=jax.ShapeDtypeStruct((B,), jnp.float32),
        mesh=mesh,
        compiler_params=pltpu.CompilerParams(needs_layout_passes=False,
                                             disable_bounds_checks=True),
        scratch_types=[
            pltpu.VMEM((bpw + L,), jnp.int32),
            pltpu.VMEM((bpw + L,), jnp.int32),
            pltpu.VMEM((2, G, K, 128), jnp.float32),
            pltpu.VMEM((2, G, K, 128), jnp.float32),
            pltpu.VMEM((bpw * K,), jnp.float32),
            pltpu.VMEM((bpw,), jnp.float32),
            pltpu.SemaphoreType.DMA,
            pltpu.SemaphoreType.DMA,
            pltpu.SemaphoreType.DMA,
            pltpu.SemaphoreType.DMA,
            pltpu.SemaphoreType.DMA,
        ],
    )
    return f(u, i, W.T, H.T)


# speedup vs baseline: 6.2109x; 1.0882x over previous
import functools

import jax
import jax.numpy as jnp
from jax import lax
from jax.experimental import pallas as pl
from jax.experimental.pallas import tpu as pltpu
from jax.experimental.pallas import tpu_sc as plsc

NC = 2
NS = 16
L = 16
NW = NC * NS
K = 16
G = 8


def _mf_body(u_hbm, i_hbm, wt_hbm, ht_hbm, out_hbm,
             uidx, iidx, bufw, bufh, prod, outv,
             sem_i, sem_w0, sem_w1, sem_h0, sem_h1):
    wid = lax.axis_index("s") * NC + lax.axis_index("c")
    bpw = u_hbm.shape[0] // NW
    base = wid * bpw
    ngrp = bpw // G

    cu = pltpu.async_copy(u_hbm.at[pl.ds(base, bpw)], uidx.at[pl.ds(0, bpw)],
                          sem_i)
    ci = pltpu.async_copy(i_hbm.at[pl.ds(base, bpw)], iidx.at[pl.ds(0, bpw)],
                          sem_i)
    cu.wait()
    ci.wait()
    uidx[pl.ds(bpw, L)] = jnp.zeros((L,), jnp.int32)
    iidx[pl.ds(bpw, L)] = jnp.zeros((L,), jnp.int32)

    lanes = lax.iota(jnp.int32, L)

    def fire(g, s, sem_w, sem_h):
        uu = uidx[pl.ds(g * G, L)]
        vv = iidx[pl.ds(g * G, L)]
        for t in range(G):
            cw = pl.multiple_of((uu[t] >> 7) * 128, 128)
            ch = pl.multiple_of((vv[t] >> 7) * 128, 128)
            pltpu.async_copy(wt_hbm.at[:, pl.ds(cw, 128)], bufw.at[s, t],
                             sem_w)
            pltpu.async_copy(ht_hbm.at[:, pl.ds(ch, 128)], bufh.at[s, t],
                             sem_h)

    def drain(s, sem_w, sem_h):
        for t in range(G):
            pltpu.make_async_copy(wt_hbm.at[:, pl.ds(0, 128)], bufw.at[s, t],
                                  sem_w).wait()
            pltpu.make_async_copy(ht_hbm.at[:, pl.ds(0, 128)], bufh.at[s, t],
                                  sem_h).wait()

    def process(g, s):
        uu = uidx[pl.ds(g * G, L)]
        vv = iidx[pl.ds(g * G, L)]
        for t in range(G):
            ou = jnp.full((L,), uu[t] & 127, jnp.int32)
            ov = jnp.full((L,), vv[t] & 127, jnp.int32)
            wcol = plsc.load_gather(bufw.at[s, t], [lanes, ou])
            hcol = plsc.load_gather(bufh.at[s, t], [lanes, ov])
            plsc.store_scatter(prod, [lanes * bpw + (g * G + t)], wcol * hcol)

    fire(0, 0, sem_w0, sem_h0)

    def body(gg, carry):
        g0 = 2 * gg
        fire(g0 + 1, 1, sem_w1, sem_h1)
        drain(0, sem_w0, sem_h0)
        process(g0, 0)

        @pl.when(gg < ngrp // 2 - 1)
        def _():
            fire(g0 + 2, 0, sem_w0, sem_h0)

        drain(1, sem_w1, sem_h1)
        process(g0 + 1, 1)
        return carry

    lax.fori_loop(0, ngrp // 2, body, 0)

    for c in range(bpw // L):
        acc = jnp.zeros((L,), jnp.float32)
        for k in range(K):
            acc += prod[pl.ds(k * bpw + c * L, L)]
        outv[pl.ds(c * L, L)] = 1.0 / (1.0 + jnp.exp(-acc))

    pltpu.sync_copy(outv, out_hbm.at[pl.ds(base, bpw)])


def kernel(x, W, H):
    B = x.shape[0]
    u = x[:, 0].astype(jnp.int32)
    i = x[:, 1].astype(jnp.int32)
    bpw = B // NW
    mesh = plsc.VectorSubcoreMesh(
        core_axis_name="c", subcore_axis_name="s",
        num_cores=NC, num_subcores=NS)
    f = pl.kernel(
        _mf_body,
        out_type=jax.ShapeDtypeStruct((B,), jnp.float32),
        mesh=mesh,
        compiler_params=pltpu.CompilerParams(needs_layout_passes=False,
                                             disable_bounds_checks=True),
        scratch_types=[
            pltpu.VMEM((bpw + L,), jnp.int32),
            pltpu.VMEM((bpw + L,), jnp.int32),
            pltpu.VMEM((2, G, K, 128), jnp.float32),
            pltpu.VMEM((2, G, K, 128), jnp.float32),
            pltpu.VMEM((bpw * K,), jnp.float32),
            pltpu.VMEM((bpw,), jnp.float32),
            pltpu.SemaphoreType.DMA,
            pltpu.SemaphoreType.DMA,
            pltpu.SemaphoreType.DMA,
            pltpu.SemaphoreType.DMA,
            pltpu.SemaphoreType.DMA,
        ],
    )
    return f(u, i, W.T, H.T)


# speedup vs baseline: 6.2391x; 1.0045x over previous
import jax
import jax.numpy as jnp
from jax import lax
from jax.experimental import pallas as pl
from jax.experimental.pallas import tpu as pltpu
from jax.experimental.pallas import tpu_sc as plsc

NC = 2
NS = 16
L = 16
NW = NC * NS
K = 16
G = 8


def _mf_body(u_hbm, i_hbm, wt_hbm, ht_hbm, out_hbm,
             uidx, iidx, bufw, bufh, prod, outv,
             sem_i, sem_w0, sem_w1, sem_h0, sem_h1):
    wid = lax.axis_index("s") * NC + lax.axis_index("c")
    bpw = u_hbm.shape[0] // NW
    base = wid * bpw
    ngrp = bpw // G

    cu = pltpu.async_copy(u_hbm.at[pl.ds(base, bpw)], uidx.at[pl.ds(0, bpw)],
                          sem_i)
    ci = pltpu.async_copy(i_hbm.at[pl.ds(base, bpw)], iidx.at[pl.ds(0, bpw)],
                          sem_i)
    cu.wait()
    ci.wait()
    uidx[pl.ds(bpw, L)] = jnp.zeros((L,), jnp.int32)
    iidx[pl.ds(bpw, L)] = jnp.zeros((L,), jnp.int32)

    lanes = lax.iota(jnp.int32, L)

    def fire(g, s, sem_w, sem_h):
        uu = uidx[pl.ds(g * G, L)]
        vv = iidx[pl.ds(g * G, L)]
        for t in range(G):
            cw = pl.multiple_of((uu[t] >> 7) * 128, 128)
            ch = pl.multiple_of((vv[t] >> 7) * 128, 128)
            pltpu.async_copy(wt_hbm.at[:, pl.ds(cw, 128)], bufw.at[s, t],
                             sem_w)
            pltpu.async_copy(ht_hbm.at[:, pl.ds(ch, 128)], bufh.at[s, t],
                             sem_h)

    def drain(s, sem_w, sem_h):
        for t in range(G):
            pltpu.make_async_copy(wt_hbm.at[:, pl.ds(0, 128)], bufw.at[s, t],
                                  sem_w).wait()
            pltpu.make_async_copy(ht_hbm.at[:, pl.ds(0, 128)], bufh.at[s, t],
                                  sem_h).wait()

    def process(g, s):
        uu = uidx[pl.ds(g * G, L)]
        vv = iidx[pl.ds(g * G, L)]
        for t in range(G):
            ou = jnp.full((L,), uu[t] & 127, jnp.int32)
            ov = jnp.full((L,), vv[t] & 127, jnp.int32)
            wcol = plsc.load_gather(bufw.at[s, t], [lanes, ou])
            hcol = plsc.load_gather(bufh.at[s, t], [lanes, ov])
            plsc.store_scatter(prod, [lanes * bpw + (g * G + t)], wcol * hcol)

    fire(0, 0, sem_w0, sem_h0)

    def body(gg, carry):
        g0 = 2 * gg
        fire(g0 + 1, 1, sem_w1, sem_h1)
        drain(0, sem_w0, sem_h0)
        process(g0, 0)

        @pl.when(gg < ngrp // 2 - 1)
        def _():
            fire(g0 + 2, 0, sem_w0, sem_h0)

        drain(1, sem_w1, sem_h1)
        process(g0 + 1, 1)
        return carry

    lax.fori_loop(0, ngrp // 2, body, 0)

    for c in range(bpw // L):
        acc = jnp.zeros((L,), jnp.float32)
        for k in range(K):
            acc += prod[pl.ds(k * bpw + c * L, L)]
        outv[pl.ds(c * L, L)] = 1.0 / (1.0 + jnp.exp(-acc))

    pltpu.sync_copy(outv, out_hbm.at[pl.ds(base, bpw)])


def kernel(x, W, H):
    B = x.shape[0]
    u = x[:, 0].astype(jnp.int32)
    i = x[:, 1].astype(jnp.int32)
    bpw = B // NW
    mesh = plsc.VectorSubcoreMesh(
        core_axis_name="c", subcore_axis_name="s",
        num_cores=NC, num_subcores=NS)
    f = pl.kernel(
        _mf_body,
        out_type=jax.ShapeDtypeStruct((B,), jnp.float32),
        mesh=mesh,
        compiler_params=pltpu.CompilerParams(needs_layout_passes=False,
                                             disable_bounds_checks=True),
        scratch_types=[
            pltpu.VMEM((bpw + L,), jnp.int32),
            pltpu.VMEM((bpw + L,), jnp.int32),
            pltpu.VMEM((2, G, K, 128), jnp.float32),
            pltpu.VMEM((2, G, K, 128), jnp.float32),
            pltpu.VMEM((bpw * K,), jnp.float32),
            pltpu.VMEM((bpw,), jnp.float32),
            pltpu.SemaphoreType.DMA,
            pltpu.SemaphoreType.DMA,
            pltpu.SemaphoreType.DMA,
            pltpu.SemaphoreType.DMA,
            pltpu.SemaphoreType.DMA,
        ],
    )
    return f(u, i, W.T, H.T)


# speedup vs baseline: 6.2416x; 1.0004x over previous
import jax
import jax.numpy as jnp
from jax import lax
from jax.experimental import pallas as pl
from jax.experimental.pallas import tpu as pltpu
from jax.experimental.pallas import tpu_sc as plsc

NC = 2
NS = 16
L = 16
NW = NC * NS
K = 16
G = 8


def _mf_body(u_hbm, i_hbm, wt_hbm, ht_hbm, out_hbm,
             uidx, iidx, bufw, bufh, prod, outv,
             sem_i, sem_w0, sem_w1, sem_h0, sem_h1):
    wid = lax.axis_index("s") * NC + lax.axis_index("c")
    bpw = u_hbm.shape[0] // NW
    base = wid * bpw
    ngrp = bpw // G

    cu = pltpu.async_copy(u_hbm.at[pl.ds(base, bpw)], uidx.at[pl.ds(0, bpw)],
                          sem_i)
    ci = pltpu.async_copy(i_hbm.at[pl.ds(base, bpw)], iidx.at[pl.ds(0, bpw)],
                          sem_i)
    cu.wait()
    ci.wait()
    uidx[pl.ds(bpw, L)] = jnp.zeros((L,), jnp.int32)
    iidx[pl.ds(bpw, L)] = jnp.zeros((L,), jnp.int32)

    lanes = lax.iota(jnp.int32, L)

    def fire(g, s, sem_w, sem_h):
        uu = uidx[pl.ds(g * G, L)]
        vv = iidx[pl.ds(g * G, L)]
        for t in range(G):
            cw = pl.multiple_of((uu[t] >> 7) * 128, 128)
            ch = pl.multiple_of((vv[t] >> 7) * 128, 128)
            for h in range(2):
                pltpu.async_copy(
                    wt_hbm.at[pl.ds(8 * h, 8), pl.ds(cw, 128)],
                    bufw.at[s, t, pl.ds(8 * h, 8)], sem_w)
                pltpu.async_copy(
                    ht_hbm.at[pl.ds(8 * h, 8), pl.ds(ch, 128)],
                    bufh.at[s, t, pl.ds(8 * h, 8)], sem_h)

    def drain(s, sem_w, sem_h):
        for t in range(G):
            pltpu.make_async_copy(wt_hbm.at[:, pl.ds(0, 128)], bufw.at[s, t],
                                  sem_w).wait()
            pltpu.make_async_copy(ht_hbm.at[:, pl.ds(0, 128)], bufh.at[s, t],
                                  sem_h).wait()

    def process(g, s):
        uu = uidx[pl.ds(g * G, L)]
        vv = iidx[pl.ds(g * G, L)]
        for t in range(G):
            ou = jnp.full((L,), uu[t] & 127, jnp.int32)
            ov = jnp.full((L,), vv[t] & 127, jnp.int32)
            wcol = plsc.load_gather(bufw.at[s, t], [lanes, ou])
            hcol = plsc.load_gather(bufh.at[s, t], [lanes, ov])
            plsc.store_scatter(prod, [lanes * bpw + (g * G + t)], wcol * hcol)

    fire(0, 0, sem_w0, sem_h0)

    def body(gg, carry):
        g0 = 2 * gg
        fire(g0 + 1, 1, sem_w1, sem_h1)
        drain(0, sem_w0, sem_h0)
        process(g0, 0)

        @pl.when(gg < ngrp // 2 - 1)
        def _():
            fire(g0 + 2, 0, sem_w0, sem_h0)

        drain(1, sem_w1, sem_h1)
        process(g0 + 1, 1)
        return carry

    lax.fori_loop(0, ngrp // 2, body, 0)

    for c in range(bpw // L):
        acc = jnp.zeros((L,), jnp.float32)
        for k in range(K):
            acc += prod[pl.ds(k * bpw + c * L, L)]
        outv[pl.ds(c * L, L)] = 1.0 / (1.0 + jnp.exp(-acc))

    pltpu.sync_copy(outv, out_hbm.at[pl.ds(base, bpw)])


def kernel(x, W, H):
    B = x.shape[0]
    u = x[:, 0].astype(jnp.int32)
    i = x[:, 1].astype(jnp.int32)
    bpw = B // NW
    mesh = plsc.VectorSubcoreMesh(
        core_axis_name="c", subcore_axis_name="s",
        num_cores=NC, num_subcores=NS)
    f = pl.kernel(
        _mf_body,
        out_type=jax.ShapeDtypeStruct((B,), jnp.float32),
        mesh=mesh,
        compiler_params=pltpu.CompilerParams(needs_layout_passes=False,
                                             disable_bounds_checks=True),
        scratch_types=[
            pltpu.VMEM((bpw + L,), jnp.int32),
            pltpu.VMEM((bpw + L,), jnp.int32),
            pltpu.VMEM((2, G, K, 128), jnp.float32),
            pltpu.VMEM((2, G, K, 128), jnp.float32),
            pltpu.VMEM((bpw * K,), jnp.float32),
            pltpu.VMEM((bpw,), jnp.float32),
            pltpu.SemaphoreType.DMA,
            pltpu.SemaphoreType.DMA,
            pltpu.SemaphoreType.DMA,
            pltpu.SemaphoreType.DMA,
            pltpu.SemaphoreType.DMA,
        ],
    )
    return f(u, i, W.T, H.T)
